# baseline (speedup 1.0000x reference)
import functools

import jax
import jax.numpy as jnp
from jax import lax
from jax.experimental import pallas as pl
from jax.experimental.pallas import tpu as pltpu
from jax.experimental.pallas import tpu_sc as plsc

_NC = 2
_NS = 16
_NW = _NC * _NS
_LANES = 16
_CH = 80
_NBUF = 5
_HIMASK = -65536


def _node_stage(x, W_i, W_c, W_o, bi, bc, bo, w_co, W1a, W1b):
    N, D = x.shape
    H = W_i.shape[1]
    TN = 1000
    grid = (N // TN,)

    def body(x_ref, wi_ref, wc_ref, wo_ref, bi_ref, bc_ref, bo_ref, wco_ref,
             w1a_ref, w1b_ref, a_ref, b_ref):
        xb = x_ref[...]
        gi = jax.nn.sigmoid(
            jnp.dot(xb, wi_ref[...], preferred_element_type=jnp.float32)
            + bi_ref[...])
        gc = jnp.tanh(
            jnp.dot(xb, wc_ref[...], preferred_element_type=jnp.float32)
            + bc_ref[...])
        c = gi * gc
        go = jax.nn.sigmoid(
            jnp.dot(xb, wo_ref[...], preferred_element_type=jnp.float32)
            + bo_ref[...] + wco_ref[...] * c)
        h = go * jnp.tanh(c)
        a_ref[...] = jnp.dot(h, w1a_ref[...], preferred_element_type=jnp.float32)
        b_ref[...] = jnp.dot(h, w1b_ref[...], preferred_element_type=jnp.float32)

    full = lambda s: pl.BlockSpec(s, lambda i: (0, 0))
    return pl.pallas_call(
        body,
        grid=grid,
        in_specs=[
            pl.BlockSpec((TN, D), lambda i: (i, 0)),
            full((D, H)), full((D, H)), full((D, H)),
            full((1, H)), full((1, H)), full((1, H)), full((1, H)),
            full((H, H)), full((H, H)),
        ],
        out_specs=[
            pl.BlockSpec((TN, H), lambda i: (i, 0)),
            pl.BlockSpec((TN, H), lambda i: (i, 0)),
        ],
        out_shape=[
            jax.ShapeDtypeStruct((N, H), jnp.float32),
            jax.ShapeDtypeStruct((N, H), jnp.float32),
        ],
    )(x, W_i, W_c, W_o, bi, bc, bo, w_co, W1a, W1b)


def _pack_bf16_halves(t):
    W = t.shape[1] // 2
    tb = t.astype(jnp.bfloat16)
    lo = lax.bitcast_convert_type(tb[:, :W], jnp.uint16).astype(jnp.uint32)
    hi = lax.bitcast_convert_type(tb[:, W:], jnp.uint16).astype(jnp.uint32)
    return lax.bitcast_convert_type(lo | (hi << 16), jnp.int32)


def _packed_add(a, b):
    f32 = lambda v: lax.bitcast_convert_type(v, jnp.float32)
    i32 = lambda v: lax.bitcast_convert_type(v, jnp.int32)
    lo = i32(f32(a << 16) + f32(b << 16))
    hi = i32(f32(a & _HIMASK) + f32(b & _HIMASK))
    lo16 = lax.shift_right_logical(lo + 0x8000, 16)
    hi16 = (hi + 0x8000) & _HIMASK
    return lo16 | hi16


def _gather_add(A_pk, B_pk, src, dst):
    E = src.shape[0]
    HW = A_pk.shape[1]
    per_w = E // _NW
    n_chunks = per_w // _CH
    n_blocks = n_chunks // _NBUF
    src3 = src.reshape(_NW, n_chunks, _CH)
    dst3 = dst.reshape(_NW, n_chunks, _CH)
    mesh = plsc.VectorSubcoreMesh(core_axis_name="c", subcore_axis_name="s")

    scratch = [
        pltpu.VMEM((n_chunks, _CH), jnp.int32),
        pltpu.VMEM((n_chunks, _CH), jnp.int32),
    ]
    scratch += [pltpu.VMEM((_CH, HW), jnp.int32) for _ in range(2 * _NBUF)]
    scratch += [pltpu.VMEM((_CH // 2, 2 * HW), jnp.int32) for _ in range(_NBUF)]
    scratch += [pltpu.SemaphoreType.DMA for _ in range(2 * _NBUF)]

    @functools.partial(
        pl.kernel,
        mesh=mesh,
        out_type=jax.ShapeDtypeStruct((E // 2, 2 * HW), jnp.int32),
        scratch_types=scratch,
        compiler_params=pltpu.CompilerParams(use_tc_tiling_on_sc=False),
    )
    def k(a_hbm, b_hbm, src_hbm, dst_hbm, g_hbm, idx_s, idx_d, *scr):
        bufA = scr[0:_NBUF]
        bufB = scr[_NBUF:2 * _NBUF]
        bufO = scr[2 * _NBUF:3 * _NBUF]
        semg = scr[3 * _NBUF:4 * _NBUF]
        sems = scr[4 * _NBUF:5 * _NBUF]

        wid = lax.axis_index("s") * _NC + lax.axis_index("c")
        w_base2 = wid * (per_w // 2)

        pltpu.sync_copy(src_hbm.at[wid], idx_s)
        pltpu.sync_copy(dst_hbm.at[wid], idx_d)

        def issue_gathers(t, b):
            pltpu.async_copy(a_hbm.at[idx_s.at[t]], bufA[b], semg[b])
            pltpu.async_copy(b_hbm.at[idx_d.at[t]], bufB[b], semg[b])

        for b in range(_NBUF):
            issue_gathers(b, b)

        def block(g, carry):
            for b in range(_NBUF):
                t = g * _NBUF + b
                pltpu.make_async_copy(
                    a_hbm.at[idx_s.at[0]], bufA[b], semg[b]).wait()
                pltpu.make_async_copy(
                    b_hbm.at[idx_d.at[0]], bufB[b], semg[b]).wait()

                @pl.when(g >= 1)
                def _():
                    pltpu.make_async_copy(
                        bufO[b], g_hbm.at[pl.ds(0, _CH // 2)], sems[b]).wait()

                def row(p, c2):
                    for u in range(2):
                        r = 2 * p + u
                        for j in range(HW // _LANES):
                            sl = pl.ds(j * _LANES, _LANES)
                            osl = pl.ds(u * HW + j * _LANES, _LANES)
                            bufO[b][p, osl] = _packed_add(
                                bufA[b][r, sl], bufB[b][r, sl])
                    return c2

                lax.fori_loop(0, _CH // 2, row, 0)

                pltpu.async_copy(
                    bufO[b],
                    g_hbm.at[pl.ds(w_base2 + t * (_CH // 2), _CH // 2)],
                    sems[b])

                @pl.when(g < n_blocks - 1)
                def _():
                    issue_gathers(t + _NBUF, b)
            return carry

        lax.fori_loop(0, n_blocks, block, 0)

        for b in range(_NBUF):
            pltpu.make_async_copy(
                bufO[b], g_hbm.at[pl.ds(0, _CH // 2)], sems[b]).wait()

    return k(A_pk, B_pk, src3, dst3)


def _interleave_out(outT):
    E2 = outT.shape[1]
    per_w = E2 // _NW
    CH3 = 1000
    n3 = per_w // CH3
    mesh = plsc.VectorSubcoreMesh(core_axis_name="c", subcore_axis_name="s")

    @functools.partial(
        pl.kernel,
        mesh=mesh,
        out_type=[
            jax.ShapeDtypeStruct((2 * E2,), jnp.float32),
            jax.ShapeDtypeStruct((2 * E2,), jnp.float32),
        ],
        scratch_types=[
            pltpu.VMEM((4, CH3), jnp.float32),
            pltpu.VMEM((2 * CH3,), jnp.float32),
            pltpu.VMEM((2 * CH3,), jnp.float32),
        ],
        compiler_params=pltpu.CompilerParams(
            use_tc_tiling_on_sc=False, needs_layout_passes=False),
    )
    def k(in_hbm, o0_hbm, o1_hbm, vin, v0, v1):
        wid = lax.axis_index("s") * _NC + lax.axis_index("c")
        pb0 = wid * per_w
        lanes = lax.iota(jnp.int32, _LANES)
        row0 = (lanes & 1) * 2
        row1 = row0 + 1
        col_base = lanes >> 1

        def chunk(t, carry):
            pb = pb0 + t * CH3
            pltpu.sync_copy(in_hbm.at[:, pl.ds(pb, CH3)], vin)

            def grp(g, c2):
                cols = col_base + g * 8
                v0[pl.ds(g * _LANES, _LANES)] = plsc.load_gather(
                    vin, [row0, cols])
                v1[pl.ds(g * _LANES, _LANES)] = plsc.load_gather(
                    vin, [row1, cols])
                return c2

            lax.fori_loop(0, (2 * CH3) // _LANES, grp, 0)
            pltpu.sync_copy(v0, o0_hbm.at[pl.ds(2 * pb, 2 * CH3)])
            pltpu.sync_copy(v1, o1_hbm.at[pl.ds(2 * pb, 2 * CH3)])
            return carry

        lax.fori_loop(0, n3, chunk, 0)

    return k(outT)


def _edge_stage(G2, attr2, W1elo, W1ehi, b1lo, b1hi, W2lo, W2hi, b2T):
    E2, W = G2.shape
    DA = attr2.shape[1]
    TE2 = 3200
    grid = (E2 // TE2,)
    attr3 = attr2.reshape(E2 // TE2, TE2, DA)

    def body(g_ref, attr_hbm, w1elo_ref, w1ehi_ref, b1lo_ref, b1hi_ref,
             w2lo_ref, w2hi_ref, b2t_ref, out_ref, abuf, asem):
        i = pl.program_id(0)
        n = pl.num_programs(0)

        @pl.when(i == 0)
        def _():
            pltpu.make_async_copy(attr_hbm.at[0], abuf.at[0], asem.at[0]).start()

        @pl.when(i + 1 < n)
        def _():
            pltpu.make_async_copy(
                attr_hbm.at[i + 1], abuf.at[(i + 1) % 2],
                asem.at[(i + 1) % 2]).start()

        pltpu.make_async_copy(
            attr_hbm.at[i], abuf.at[i % 2], asem.at[i % 2]).wait()

        bits = g_ref[...]
        glo = lax.bitcast_convert_type(bits << 16, jnp.float32)
        ghi = lax.bitcast_convert_type(bits & _HIMASK, jnp.float32)
        at = abuf[i % 2].astype(jnp.bfloat16)
        elo = jnp.dot(at, w1elo_ref[...], preferred_element_type=jnp.float32)
        ehi = jnp.dot(at, w1ehi_ref[...], preferred_element_type=jnp.float32)
        hid_lo = jnp.maximum(glo + elo + b1lo_ref[...], 0.0).astype(jnp.bfloat16)
        hid_hi = jnp.maximum(ghi + ehi + b1hi_ref[...], 0.0).astype(jnp.bfloat16)
        olo = lax.dot_general(w2lo_ref[...], hid_lo, (((0,), (1,)), ((), ())),
                              preferred_element_type=jnp.float32)
        ohi = lax.dot_general(w2hi_ref[...], hid_hi, (((0,), (1,)), ((), ())),
                              preferred_element_type=jnp.float32)
        out_ref[...] = olo + ohi + b2t_ref[...]

    full = lambda s: pl.BlockSpec(s, lambda i: (0, 0))
    return pl.pallas_call(
        body,
        grid=grid,
        in_specs=[
            pl.BlockSpec((TE2, W), lambda i: (i, 0)),
            pl.BlockSpec(memory_space=pltpu.MemorySpace.HBM),
            full(W1elo.shape), full(W1ehi.shape),
            full(b1lo.shape), full(b1hi.shape),
            full(W2lo.shape), full(W2hi.shape), full(b2T.shape),
        ],
        out_specs=pl.BlockSpec((4, TE2), lambda i: (0, i)),
        out_shape=jax.ShapeDtypeStruct((4, E2), jnp.float32),
        scratch_shapes=[
            pltpu.VMEM((2, TE2, DA), jnp.float32),
            pltpu.SemaphoreType.DMA((2,)),
        ],
    )(G2, attr3, W1elo, W1ehi, b1lo, b1hi, W2lo, W2hi, b2T)


def kernel(x, edge_index, edge_attr, W_i, W_f, W_c, W_o, b_i, b_f, b_c, b_o,
           w_ci, w_cf, w_co, T_i, T_f, T_c, T_o, cb_i, cb_f, cb_c, cb_o,
           W1, b1, W2, b2):
    H = W_i.shape[1]
    Hh = H // 2
    E = edge_index.shape[1]
    bi = b_i + cb_i[None, :]
    bc = b_c + cb_c[None, :]
    bo = b_o + cb_o[None, :]
    W1a = W1[:H]
    W1b = W1[H:2 * H]
    W1e = W1[2 * H:]
    DE = W1e.shape[0]

    Z = jnp.zeros((DE, Hh), dtype=W1e.dtype)
    W1elo = jnp.concatenate([
        jnp.concatenate([W1e[:, :Hh], Z], axis=1),
        jnp.concatenate([Z, W1e[:, :Hh]], axis=1)], axis=0).astype(jnp.bfloat16)
    W1ehi = jnp.concatenate([
        jnp.concatenate([W1e[:, Hh:], Z], axis=1),
        jnp.concatenate([Z, W1e[:, Hh:]], axis=1)], axis=0).astype(jnp.bfloat16)
    b1lo = jnp.concatenate([b1[:Hh], b1[:Hh]])[None, :]
    b1hi = jnp.concatenate([b1[Hh:], b1[Hh:]])[None, :]
    C = W2.shape[1]
    Z2 = jnp.zeros((Hh, C), dtype=W2.dtype)
    W2lo = jnp.concatenate([
        jnp.concatenate([W2[:Hh], Z2], axis=1),
        jnp.concatenate([Z2, W2[:Hh]], axis=1)], axis=0).astype(jnp.bfloat16)
    W2hi = jnp.concatenate([
        jnp.concatenate([W2[Hh:], Z2], axis=1),
        jnp.concatenate([Z2, W2[Hh:]], axis=1)], axis=0).astype(jnp.bfloat16)
    b2T = jnp.concatenate([b2, b2])[:, None]

    A, B = _node_stage(x, W_i, W_c, W_o, bi, bc, bo, w_co, W1a, W1b)
    A_pk = _pack_bf16_halves(A)
    B_pk = _pack_bf16_halves(B)
    G2 = _gather_add(A_pk, B_pk, edge_index[0], edge_index[1])
    attr2 = edge_attr.reshape(E // 2, 2 * DE)
    outT = _edge_stage(G2, attr2, W1elo, W1ehi, b1lo, b1hi, W2lo, W2hi, b2T)
    o0, o1 = _interleave_out(outT)
    return jnp.concatenate([o0[:, None], o1[:, None]], axis=1)


# speedup vs baseline: 1.0138x; 1.0138x over previous
import functools

import jax
import jax.numpy as jnp
from jax import lax
from jax.experimental import pallas as pl
from jax.experimental.pallas import tpu as pltpu
from jax.experimental.pallas import tpu_sc as plsc

_NC = 2
_NS = 16
_NW = _NC * _NS
_LANES = 16
_CH = 80
_NBUF = 5
_HIMASK = -65536


def _node_stage(x, W_i, W_c, W_o, bi, bc, bo, w_co, W1a, W1b):
    N, D = x.shape
    H = W_i.shape[1]
    TN = 1000
    grid = (N // TN,)

    def body(x_ref, wi_ref, wc_ref, wo_ref, bi_ref, bc_ref, bo_ref, wco_ref,
             w1a_ref, w1b_ref, a_ref, b_ref):
        xb = x_ref[...]
        gi = jax.nn.sigmoid(
            jnp.dot(xb, wi_ref[...], preferred_element_type=jnp.float32)
            + bi_ref[...])
        gc = jnp.tanh(
            jnp.dot(xb, wc_ref[...], preferred_element_type=jnp.float32)
            + bc_ref[...])
        c = gi * gc
        go = jax.nn.sigmoid(
            jnp.dot(xb, wo_ref[...], preferred_element_type=jnp.float32)
            + bo_ref[...] + wco_ref[...] * c)
        h = go * jnp.tanh(c)
        az = jnp.dot(h, w1a_ref[...], preferred_element_type=jnp.float32)
        bz = jnp.dot(h, w1b_ref[...], preferred_element_type=jnp.float32)
        a_ref[...] = _pack_tc(az)
        b_ref[...] = _pack_tc(bz)

    full = lambda s: pl.BlockSpec(s, lambda i: (0, 0))
    return pl.pallas_call(
        body,
        grid=grid,
        in_specs=[
            pl.BlockSpec((TN, D), lambda i: (i, 0)),
            full((D, H)), full((D, H)), full((D, H)),
            full((1, H)), full((1, H)), full((1, H)), full((1, H)),
            full((H, H)), full((H, H)),
        ],
        out_specs=[
            pl.BlockSpec((TN, H // 2), lambda i: (i, 0)),
            pl.BlockSpec((TN, H // 2), lambda i: (i, 0)),
        ],
        out_shape=[
            jax.ShapeDtypeStruct((N, H // 2), jnp.int32),
            jax.ShapeDtypeStruct((N, H // 2), jnp.int32),
        ],
    )(x, W_i, W_c, W_o, bi, bc, bo, w_co, W1a, W1b)


def _pack_tc(t):
    W = t.shape[1] // 2
    lo = lax.bitcast_convert_type(t[:, :W], jnp.int32)
    hi = lax.bitcast_convert_type(t[:, W:], jnp.int32)
    lo16 = lax.shift_right_logical(lo + 0x8000, 16)
    hi16 = (hi + 0x8000) & _HIMASK
    return lo16 | hi16


def _packed_add(a, b):
    f32 = lambda v: lax.bitcast_convert_type(v, jnp.float32)
    i32 = lambda v: lax.bitcast_convert_type(v, jnp.int32)
    lo = i32(f32(a << 16) + f32(b << 16))
    hi = i32(f32(a & _HIMASK) + f32(b & _HIMASK))
    lo16 = lax.shift_right_logical(lo + 0x8000, 16)
    hi16 = (hi + 0x8000) & _HIMASK
    return lo16 | hi16


def _gather_add(A_pk, B_pk, src, dst):
    E = src.shape[0]
    HW = A_pk.shape[1]
    per_w = E // _NW
    n_chunks = per_w // _CH
    n_blocks = n_chunks // _NBUF
    mesh = plsc.VectorSubcoreMesh(core_axis_name="c", subcore_axis_name="s")

    scratch = [
        pltpu.VMEM((per_w,), jnp.int32),
        pltpu.VMEM((per_w,), jnp.int32),
    ]
    scratch += [pltpu.VMEM((_CH, HW), jnp.int32) for _ in range(2 * _NBUF)]
    scratch += [pltpu.VMEM((_CH // 2, 2 * HW), jnp.int32) for _ in range(_NBUF)]
    scratch += [pltpu.SemaphoreType.DMA for _ in range(2 * _NBUF)]

    @functools.partial(
        pl.kernel,
        mesh=mesh,
        out_type=jax.ShapeDtypeStruct((E // 2, 2 * HW), jnp.int32),
        scratch_types=scratch,
        compiler_params=pltpu.CompilerParams(use_tc_tiling_on_sc=False),
    )
    def k(a_hbm, b_hbm, src_hbm, dst_hbm, g_hbm, idx_s, idx_d, *scr):
        bufA = scr[0:_NBUF]
        bufB = scr[_NBUF:2 * _NBUF]
        bufO = scr[2 * _NBUF:3 * _NBUF]
        semg = scr[3 * _NBUF:4 * _NBUF]
        sems = scr[4 * _NBUF:5 * _NBUF]

        wid = lax.axis_index("s") * _NC + lax.axis_index("c")
        w_base2 = wid * (per_w // 2)

        pltpu.sync_copy(src_hbm.at[pl.ds(wid * per_w, per_w)], idx_s)
        pltpu.sync_copy(dst_hbm.at[pl.ds(wid * per_w, per_w)], idx_d)

        def issue_gathers(t, b):
            pltpu.async_copy(
                a_hbm.at[idx_s.at[pl.ds(t * _CH, _CH)]], bufA[b], semg[b])
            pltpu.async_copy(
                b_hbm.at[idx_d.at[pl.ds(t * _CH, _CH)]], bufB[b], semg[b])

        for b in range(_NBUF):
            issue_gathers(b, b)

        def block(g, carry):
            for b in range(_NBUF):
                t = g * _NBUF + b
                pltpu.make_async_copy(
                    a_hbm.at[idx_s.at[pl.ds(0, _CH)]], bufA[b], semg[b]).wait()
                pltpu.make_async_copy(
                    b_hbm.at[idx_d.at[pl.ds(0, _CH)]], bufB[b], semg[b]).wait()

                @pl.when(g >= 1)
                def _():
                    pltpu.make_async_copy(
                        bufO[b], g_hbm.at[pl.ds(0, _CH // 2)], sems[b]).wait()

                def row(p, c2):
                    for u in range(2):
                        r = 2 * p + u
                        for j in range(HW // _LANES):
                            sl = pl.ds(j * _LANES, _LANES)
                            osl = pl.ds(u * HW + j * _LANES, _LANES)
                            bufO[b][p, osl] = _packed_add(
                                bufA[b][r, sl], bufB[b][r, sl])
                    return c2

                lax.fori_loop(0, _CH // 2, row, 0)

                pltpu.async_copy(
                    bufO[b],
                    g_hbm.at[pl.ds(w_base2 + t * (_CH // 2), _CH // 2)],
                    sems[b])

                @pl.when(g < n_blocks - 1)
                def _():
                    issue_gathers(t + _NBUF, b)
            return carry

        lax.fori_loop(0, n_blocks, block, 0)

        for b in range(_NBUF):
            pltpu.make_async_copy(
                bufO[b], g_hbm.at[pl.ds(0, _CH // 2)], sems[b]).wait()

    return k(A_pk, B_pk, src, dst)


def _interleave_out(outT):
    E2 = outT.shape[1]
    per_w = E2 // _NW
    CH3 = 1000
    n3 = per_w // CH3
    mesh = plsc.VectorSubcoreMesh(core_axis_name="c", subcore_axis_name="s")

    @functools.partial(
        pl.kernel,
        mesh=mesh,
        out_type=[
            jax.ShapeDtypeStruct((2 * E2,), jnp.float32),
            jax.ShapeDtypeStruct((2 * E2,), jnp.float32),
        ],
        scratch_types=[
            pltpu.VMEM((4, CH3), jnp.float32),
            pltpu.VMEM((2 * CH3,), jnp.float32),
            pltpu.VMEM((2 * CH3,), jnp.float32),
        ],
        compiler_params=pltpu.CompilerParams(
            use_tc_tiling_on_sc=False, needs_layout_passes=False),
    )
    def k(in_hbm, o0_hbm, o1_hbm, vin, v0, v1):
        wid = lax.axis_index("s") * _NC + lax.axis_index("c")
        pb0 = wid * per_w
        lanes = lax.iota(jnp.int32, _LANES)
        row0 = (lanes & 1) * 2
        row1 = row0 + 1
        col_base = lanes >> 1

        def chunk(t, carry):
            pb = pb0 + t * CH3
            pltpu.sync_copy(in_hbm.at[:, pl.ds(pb, CH3)], vin)

            def grp(g, c2):
                cols = col_base + g * 8
                v0[pl.ds(g * _LANES, _LANES)] = plsc.load_gather(
                    vin, [row0, cols])
                v1[pl.ds(g * _LANES, _LANES)] = plsc.load_gather(
                    vin, [row1, cols])
                return c2

            lax.fori_loop(0, (2 * CH3) // _LANES, grp, 0)
            pltpu.sync_copy(v0, o0_hbm.at[pl.ds(2 * pb, 2 * CH3)])
            pltpu.sync_copy(v1, o1_hbm.at[pl.ds(2 * pb, 2 * CH3)])
            return carry

        lax.fori_loop(0, n3, chunk, 0)

    return k(outT)


def _edge_stage(G2, attr2, W1elo, W1ehi, b1lo, b1hi, W2lo, W2hi, b2T):
    E2, W = G2.shape
    DA = attr2.shape[1]
    TE2 = 3200
    grid = (E2 // TE2,)
    attr3 = attr2.reshape(E2 // TE2, TE2, DA)

    def body(g_ref, attr_hbm, w1elo_ref, w1ehi_ref, b1lo_ref, b1hi_ref,
             w2lo_ref, w2hi_ref, b2t_ref, out_ref, abuf, asem):
        i = pl.program_id(0)
        n = pl.num_programs(0)

        @pl.when(i == 0)
        def _():
            pltpu.make_async_copy(attr_hbm.at[0], abuf.at[0], asem.at[0]).start()

        @pl.when(i + 1 < n)
        def _():
            pltpu.make_async_copy(
                attr_hbm.at[i + 1], abuf.at[(i + 1) % 2],
                asem.at[(i + 1) % 2]).start()

        pltpu.make_async_copy(
            attr_hbm.at[i], abuf.at[i % 2], asem.at[i % 2]).wait()

        bits = g_ref[...]
        glo = lax.bitcast_convert_type(bits << 16, jnp.float32)
        ghi = lax.bitcast_convert_type(bits & _HIMASK, jnp.float32)
        at = abuf[i % 2].astype(jnp.bfloat16)
        elo = jnp.dot(at, w1elo_ref[...], preferred_element_type=jnp.float32)
        ehi = jnp.dot(at, w1ehi_ref[...], preferred_element_type=jnp.float32)
        hid_lo = jnp.maximum(glo + elo + b1lo_ref[...], 0.0).astype(jnp.bfloat16)
        hid_hi = jnp.maximum(ghi + ehi + b1hi_ref[...], 0.0).astype(jnp.bfloat16)
        olo = lax.dot_general(w2lo_ref[...], hid_lo, (((0,), (1,)), ((), ())),
                              preferred_element_type=jnp.float32)
        ohi = lax.dot_general(w2hi_ref[...], hid_hi, (((0,), (1,)), ((), ())),
                              preferred_element_type=jnp.float32)
        out_ref[...] = olo + ohi + b2t_ref[...]

    full = lambda s: pl.BlockSpec(s, lambda i: (0, 0))
    return pl.pallas_call(
        body,
        grid=grid,
        in_specs=[
            pl.BlockSpec((TE2, W), lambda i: (i, 0)),
            pl.BlockSpec(memory_space=pltpu.MemorySpace.HBM),
            full(W1elo.shape), full(W1ehi.shape),
            full(b1lo.shape), full(b1hi.shape),
            full(W2lo.shape), full(W2hi.shape), full(b2T.shape),
        ],
        out_specs=pl.BlockSpec((4, TE2), lambda i: (0, i)),
        out_shape=jax.ShapeDtypeStruct((4, E2), jnp.float32),
        scratch_shapes=[
            pltpu.VMEM((2, TE2, DA), jnp.float32),
            pltpu.SemaphoreType.DMA((2,)),
        ],
    )(G2, attr3, W1elo, W1ehi, b1lo, b1hi, W2lo, W2hi, b2T)


def kernel(x, edge_index, edge_attr, W_i, W_f, W_c, W_o, b_i, b_f, b_c, b_o,
           w_ci, w_cf, w_co, T_i, T_f, T_c, T_o, cb_i, cb_f, cb_c, cb_o,
           W1, b1, W2, b2):
    H = W_i.shape[1]
    Hh = H // 2
    E = edge_index.shape[1]
    bi = b_i + cb_i[None, :]
    bc = b_c + cb_c[None, :]
    bo = b_o + cb_o[None, :]
    W1a = W1[:H]
    W1b = W1[H:2 * H]
    W1e = W1[2 * H:]
    DE = W1e.shape[0]

    Z = jnp.zeros((DE, Hh), dtype=W1e.dtype)
    W1elo = jnp.concatenate([
        jnp.concatenate([W1e[:, :Hh], Z], axis=1),
        jnp.concatenate([Z, W1e[:, :Hh]], axis=1)], axis=0).astype(jnp.bfloat16)
    W1ehi = jnp.concatenate([
        jnp.concatenate([W1e[:, Hh:], Z], axis=1),
        jnp.concatenate([Z, W1e[:, Hh:]], axis=1)], axis=0).astype(jnp.bfloat16)
    b1lo = jnp.concatenate([b1[:Hh], b1[:Hh]])[None, :]
    b1hi = jnp.concatenate([b1[Hh:], b1[Hh:]])[None, :]
    C = W2.shape[1]
    Z2 = jnp.zeros((Hh, C), dtype=W2.dtype)
    W2lo = jnp.concatenate([
        jnp.concatenate([W2[:Hh], Z2], axis=1),
        jnp.concatenate([Z2, W2[:Hh]], axis=1)], axis=0).astype(jnp.bfloat16)
    W2hi = jnp.concatenate([
        jnp.concatenate([W2[Hh:], Z2], axis=1),
        jnp.concatenate([Z2, W2[Hh:]], axis=1)], axis=0).astype(jnp.bfloat16)
    b2T = jnp.concatenate([b2, b2])[:, None]

    A_pk, B_pk = _node_stage(x, W_i, W_c, W_o, bi, bc, bo, w_co, W1a, W1b)
    G2 = _gather_add(A_pk, B_pk, edge_index[0], edge_index[1])
    attr2 = edge_attr.reshape(E // 2, 2 * DE)
    outT = _edge_stage(G2, attr2, W1elo, W1ehi, b1lo, b1hi, W2lo, W2hi, b2T)
    o0, o1 = _interleave_out(outT)
    return jnp.concatenate([o0[:, None], o1[:, None]], axis=1)


# speedup vs baseline: 1.0525x; 1.0381x over previous
import functools

import jax
import jax.numpy as jnp
from jax import lax
from jax.experimental import pallas as pl
from jax.experimental.pallas import tpu as pltpu
from jax.experimental.pallas import tpu_sc as plsc

_NC = 2
_NS = 16
_NW = _NC * _NS
_LANES = 16
_CH = 80
_NBUF = 5
_HIMASK = -65536


def _node_stage(x, W_i, W_c, W_o, bi, bc, bo, w_co, W1a, W1b):
    N, D = x.shape
    H = W_i.shape[1]
    TN = 1000
    grid = (N // TN,)

    def body(x_ref, wi_ref, wc_ref, wo_ref, bi_ref, bc_ref, bo_ref, wco_ref,
             w1a_ref, w1b_ref, a_ref, b_ref):
        xb = x_ref[...]
        gi = jax.nn.sigmoid(
            jnp.dot(xb, wi_ref[...], preferred_element_type=jnp.float32)
            + bi_ref[...])
        gc = jnp.tanh(
            jnp.dot(xb, wc_ref[...], preferred_element_type=jnp.float32)
            + bc_ref[...])
        c = gi * gc
        go = jax.nn.sigmoid(
            jnp.dot(xb, wo_ref[...], preferred_element_type=jnp.float32)
            + bo_ref[...] + wco_ref[...] * c)
        h = go * jnp.tanh(c)
        az = jnp.dot(h, w1a_ref[...], preferred_element_type=jnp.float32)
        bz = jnp.dot(h, w1b_ref[...], preferred_element_type=jnp.float32)
        a_ref[...] = _pack_tc(az)
        b_ref[...] = _pack_tc(bz)

    full = lambda s: pl.BlockSpec(s, lambda i: (0, 0))
    return pl.pallas_call(
        body,
        grid=grid,
        in_specs=[
            pl.BlockSpec((TN, D), lambda i: (i, 0)),
            full((D, H)), full((D, H)), full((D, H)),
            full((1, H)), full((1, H)), full((1, H)), full((1, H)),
            full((H, H)), full((H, H)),
        ],
        out_specs=[
            pl.BlockSpec((TN, H // 2), lambda i: (i, 0)),
            pl.BlockSpec((TN, H // 2), lambda i: (i, 0)),
        ],
        out_shape=[
            jax.ShapeDtypeStruct((N, H // 2), jnp.int32),
            jax.ShapeDtypeStruct((N, H // 2), jnp.int32),
        ],
    )(x, W_i, W_c, W_o, bi, bc, bo, w_co, W1a, W1b)


def _pack_tc(t):
    W = t.shape[1] // 2
    lo = lax.bitcast_convert_type(t[:, :W], jnp.int32)
    hi = lax.bitcast_convert_type(t[:, W:], jnp.int32)
    lo16 = lax.shift_right_logical(lo + 0x8000, 16)
    hi16 = (hi + 0x8000) & _HIMASK
    return lo16 | hi16


def _packed_add(a, b):
    f32 = lambda v: lax.bitcast_convert_type(v, jnp.float32)
    i32 = lambda v: lax.bitcast_convert_type(v, jnp.int32)
    lo = i32(f32(a << 16) + f32(b << 16))
    hi = i32(f32(a & _HIMASK) + f32(b & _HIMASK))
    lo16 = lax.shift_right_logical(lo + 0x8000, 16)
    hi16 = (hi + 0x8000) & _HIMASK
    return lo16 | hi16


def _gather_add(A_pk, B_pk, src, dst):
    E = src.shape[0]
    HW = A_pk.shape[1]
    per_w = E // _NW
    n_chunks = per_w // _CH
    n_blocks = n_chunks // _NBUF
    mesh = plsc.VectorSubcoreMesh(core_axis_name="c", subcore_axis_name="s")

    scratch = [
        pltpu.VMEM((per_w,), jnp.int32),
        pltpu.VMEM((per_w,), jnp.int32),
    ]
    scratch += [pltpu.VMEM((_CH, HW), jnp.int32) for _ in range(2 * _NBUF)]
    scratch += [pltpu.VMEM((_CH // 2, 2 * HW), jnp.int32) for _ in range(_NBUF)]
    scratch += [pltpu.SemaphoreType.DMA for _ in range(2 * _NBUF)]

    @functools.partial(
        pl.kernel,
        mesh=mesh,
        out_type=jax.ShapeDtypeStruct((E // 2, 2 * HW), jnp.int32),
        scratch_types=scratch,
        compiler_params=pltpu.CompilerParams(use_tc_tiling_on_sc=False),
    )
    def k(a_hbm, b_hbm, src_hbm, dst_hbm, g_hbm, idx_s, idx_d, *scr):
        bufA = scr[0:_NBUF]
        bufB = scr[_NBUF:2 * _NBUF]
        bufO = scr[2 * _NBUF:3 * _NBUF]
        semg = scr[3 * _NBUF:4 * _NBUF]
        sems = scr[4 * _NBUF:5 * _NBUF]

        wid = lax.axis_index("s") * _NC + lax.axis_index("c")
        w_base2 = wid * (per_w // 2)

        pltpu.sync_copy(src_hbm.at[pl.ds(wid * per_w, per_w)], idx_s)
        pltpu.sync_copy(dst_hbm.at[pl.ds(wid * per_w, per_w)], idx_d)

        def issue_gathers(t, b):
            pltpu.async_copy(
                a_hbm.at[idx_s.at[pl.ds(t * _CH, _CH)]], bufA[b], semg[b])
            pltpu.async_copy(
                b_hbm.at[idx_d.at[pl.ds(t * _CH, _CH)]], bufB[b], semg[b])

        for b in range(_NBUF):
            issue_gathers(b, b)

        def block(g, carry):
            for b in range(_NBUF):
                t = g * _NBUF + b
                pltpu.make_async_copy(
                    a_hbm.at[idx_s.at[pl.ds(0, _CH)]], bufA[b], semg[b]).wait()
                pltpu.make_async_copy(
                    b_hbm.at[idx_d.at[pl.ds(0, _CH)]], bufB[b], semg[b]).wait()

                @pl.when(g >= 1)
                def _():
                    pltpu.make_async_copy(
                        bufO[b], g_hbm.at[pl.ds(0, _CH // 2)], sems[b]).wait()

                def row(p, c2):
                    for u in range(2):
                        r = 2 * p + u
                        for j in range(HW // _LANES):
                            sl = pl.ds(j * _LANES, _LANES)
                            osl = pl.ds(u * HW + j * _LANES, _LANES)
                            bufO[b][p, osl] = _packed_add(
                                bufA[b][r, sl], bufB[b][r, sl])
                    return c2

                lax.fori_loop(0, _CH // 2, row, 0)

                pltpu.async_copy(
                    bufO[b],
                    g_hbm.at[pl.ds(w_base2 + t * (_CH // 2), _CH // 2)],
                    sems[b])

                @pl.when(g < n_blocks - 1)
                def _():
                    issue_gathers(t + _NBUF, b)
            return carry

        lax.fori_loop(0, n_blocks, block, 0)

        for b in range(_NBUF):
            pltpu.make_async_copy(
                bufO[b], g_hbm.at[pl.ds(0, _CH // 2)], sems[b]).wait()

    return k(A_pk, B_pk, src, dst)


def _interleave_out(outT):
    E2 = outT.shape[1]
    per_w = E2 // _NW
    CH3 = 1000
    n3 = per_w // CH3
    mesh = plsc.VectorSubcoreMesh(core_axis_name="c", subcore_axis_name="s")

    @functools.partial(
        pl.kernel,
        mesh=mesh,
        out_type=[
            jax.ShapeDtypeStruct((2 * E2,), jnp.float32),
            jax.ShapeDtypeStruct((2 * E2,), jnp.float32),
        ],
        scratch_types=[
            pltpu.VMEM((4, CH3), jnp.float32),
            pltpu.VMEM((2 * CH3,), jnp.float32),
            pltpu.VMEM((2 * CH3,), jnp.float32),
        ],
        compiler_params=pltpu.CompilerParams(
            use_tc_tiling_on_sc=False, needs_layout_passes=False),
    )
    def k(in_hbm, o0_hbm, o1_hbm, vin, v0, v1):
        wid = lax.axis_index("s") * _NC + lax.axis_index("c")
        pb0 = wid * per_w
        lanes = lax.iota(jnp.int32, _LANES)
        row0 = (lanes & 1) * 2
        row1 = row0 + 1
        col_base = lanes >> 1

        def chunk(t, carry):
            pb = pb0 + t * CH3
            pltpu.sync_copy(in_hbm.at[:, pl.ds(pb, CH3)], vin)

            def grp(g, c2):
                cols = col_base + g * 8
                v0[pl.ds(g * _LANES, _LANES)] = plsc.load_gather(
                    vin, [row0, cols])
                v1[pl.ds(g * _LANES, _LANES)] = plsc.load_gather(
                    vin, [row1, cols])
                return c2

            lax.fori_loop(0, (2 * CH3) // _LANES, grp, 0)
            pltpu.sync_copy(v0, o0_hbm.at[pl.ds(2 * pb, 2 * CH3)])
            pltpu.sync_copy(v1, o1_hbm.at[pl.ds(2 * pb, 2 * CH3)])
            return carry

        lax.fori_loop(0, n3, chunk, 0)

    return k(outT)


def _edge_stage(G2, attr2, W1elo, W1ehi, b1lo, b1hi, W2lo, W2hi, b2T):
    E2, W = G2.shape
    DA = attr2.shape[1]
    TE2 = 6400
    grid = (E2 // TE2,)
    attr3 = attr2.reshape(E2 // TE2, TE2, DA)

    def body(g_ref, attr_hbm, w1elo_ref, w1ehi_ref, b1lo_ref, b1hi_ref,
             w2lo_ref, w2hi_ref, b2t_ref, out_ref, abuf, asem):
        i = pl.program_id(0)
        n = pl.num_programs(0)

        @pl.when(i == 0)
        def _():
            pltpu.make_async_copy(attr_hbm.at[0], abuf.at[0], asem.at[0]).start()

        @pl.when(i + 1 < n)
        def _():
            pltpu.make_async_copy(
                attr_hbm.at[i + 1], abuf.at[(i + 1) % 2],
                asem.at[(i + 1) % 2]).start()

        pltpu.make_async_copy(
            attr_hbm.at[i], abuf.at[i % 2], asem.at[i % 2]).wait()

        bits = g_ref[...]
        glo = lax.bitcast_convert_type(bits << 16, jnp.float32)
        ghi = lax.bitcast_convert_type(bits & _HIMASK, jnp.float32)
        at = abuf[i % 2].astype(jnp.bfloat16)
        elo = jnp.dot(at, w1elo_ref[...], preferred_element_type=jnp.float32)
        ehi = jnp.dot(at, w1ehi_ref[...], preferred_element_type=jnp.float32)
        hid_lo = jnp.maximum(glo + elo + b1lo_ref[...], 0.0).astype(jnp.bfloat16)
        hid_hi = jnp.maximum(ghi + ehi + b1hi_ref[...], 0.0).astype(jnp.bfloat16)
        olo = lax.dot_general(w2lo_ref[...], hid_lo, (((0,), (1,)), ((), ())),
                              preferred_element_type=jnp.float32)
        ohi = lax.dot_general(w2hi_ref[...], hid_hi, (((0,), (1,)), ((), ())),
                              preferred_element_type=jnp.float32)
        out_ref[...] = olo + ohi + b2t_ref[...]

    full = lambda s: pl.BlockSpec(s, lambda i: (0, 0))
    return pl.pallas_call(
        body,
        grid=grid,
        in_specs=[
            pl.BlockSpec((TE2, W), lambda i: (i, 0)),
            pl.BlockSpec(memory_space=pltpu.MemorySpace.HBM),
            full(W1elo.shape), full(W1ehi.shape),
            full(b1lo.shape), full(b1hi.shape),
            full(W2lo.shape), full(W2hi.shape), full(b2T.shape),
        ],
        out_specs=pl.BlockSpec((4, TE2), lambda i: (0, i)),
        out_shape=jax.ShapeDtypeStruct((4, E2), jnp.float32),
        scratch_shapes=[
            pltpu.VMEM((2, TE2, DA), jnp.float32),
            pltpu.SemaphoreType.DMA((2,)),
        ],
    )(G2, attr3, W1elo, W1ehi, b1lo, b1hi, W2lo, W2hi, b2T)


def kernel(x, edge_index, edge_attr, W_i, W_f, W_c, W_o, b_i, b_f, b_c, b_o,
           w_ci, w_cf, w_co, T_i, T_f, T_c, T_o, cb_i, cb_f, cb_c, cb_o,
           W1, b1, W2, b2):
    H = W_i.shape[1]
    Hh = H // 2
    E = edge_index.shape[1]
    bi = b_i + cb_i[None, :]
    bc = b_c + cb_c[None, :]
    bo = b_o + cb_o[None, :]
    W1a = W1[:H]
    W1b = W1[H:2 * H]
    W1e = W1[2 * H:]
    DE = W1e.shape[0]

    Z = jnp.zeros((DE, Hh), dtype=W1e.dtype)
    W1elo = jnp.concatenate([
        jnp.concatenate([W1e[:, :Hh], Z], axis=1),
        jnp.concatenate([Z, W1e[:, :Hh]], axis=1)], axis=0).astype(jnp.bfloat16)
    W1ehi = jnp.concatenate([
        jnp.concatenate([W1e[:, Hh:], Z], axis=1),
        jnp.concatenate([Z, W1e[:, Hh:]], axis=1)], axis=0).astype(jnp.bfloat16)
    b1lo = jnp.concatenate([b1[:Hh], b1[:Hh]])[None, :]
    b1hi = jnp.concatenate([b1[Hh:], b1[Hh:]])[None, :]
    C = W2.shape[1]
    Z2 = jnp.zeros((Hh, C), dtype=W2.dtype)
    W2lo = jnp.concatenate([
        jnp.concatenate([W2[:Hh], Z2], axis=1),
        jnp.concatenate([Z2, W2[:Hh]], axis=1)], axis=0).astype(jnp.bfloat16)
    W2hi = jnp.concatenate([
        jnp.concatenate([W2[Hh:], Z2], axis=1),
        jnp.concatenate([Z2, W2[Hh:]], axis=1)], axis=0).astype(jnp.bfloat16)
    b2T = jnp.concatenate([b2, b2])[:, None]

    A_pk, B_pk = _node_stage(x, W_i, W_c, W_o, bi, bc, bo, w_co, W1a, W1b)
    G2 = _gather_add(A_pk, B_pk, edge_index[0], edge_index[1])
    attr2 = edge_attr.reshape(E // 2, 2 * DE)
    outT = _edge_stage(G2, attr2, W1elo, W1ehi, b1lo, b1hi, W2lo, W2hi, b2T)
    o0, o1 = _interleave_out(outT)
    return jnp.concatenate([o0[:, None], o1[:, None]], axis=1)


# speedup vs baseline: 1.0718x; 1.0183x over previous
import functools

import jax
import jax.numpy as jnp
from jax import lax
from jax.experimental import pallas as pl
from jax.experimental.pallas import tpu as pltpu
from jax.experimental.pallas import tpu_sc as plsc

_NC = 2
_NS = 16
_NW = _NC * _NS
_LANES = 16
_CH = 80
_NBUF = 5
_HIMASK = -65536


def _node_stage(x, W_i, W_c, W_o, bi, bc, bo, w_co, W1a, W1b):
    N, D = x.shape
    H = W_i.shape[1]
    TN = 1000
    grid = (N // TN,)

    def body(x_ref, wi_ref, wc_ref, wo_ref, bi_ref, bc_ref, bo_ref, wco_ref,
             w1a_ref, w1b_ref, a_ref, b_ref):
        xb = x_ref[...]
        gi = jax.nn.sigmoid(
            jnp.dot(xb, wi_ref[...], preferred_element_type=jnp.float32)
            + bi_ref[...])
        gc = jnp.tanh(
            jnp.dot(xb, wc_ref[...], preferred_element_type=jnp.float32)
            + bc_ref[...])
        c = gi * gc
        go = jax.nn.sigmoid(
            jnp.dot(xb, wo_ref[...], preferred_element_type=jnp.float32)
            + bo_ref[...] + wco_ref[...] * c)
        h = go * jnp.tanh(c)
        az = jnp.dot(h, w1a_ref[...], preferred_element_type=jnp.float32)
        bz = jnp.dot(h, w1b_ref[...], preferred_element_type=jnp.float32)
        a_ref[...] = _pack_tc(az)
        b_ref[...] = _pack_tc(bz)

    full = lambda s: pl.BlockSpec(s, lambda i: (0, 0))
    return pl.pallas_call(
        body,
        grid=grid,
        in_specs=[
            pl.BlockSpec((TN, D), lambda i: (i, 0)),
            full((D, H)), full((D, H)), full((D, H)),
            full((1, H)), full((1, H)), full((1, H)), full((1, H)),
            full((H, H)), full((H, H)),
        ],
        out_specs=[
            pl.BlockSpec((TN, H // 2), lambda i: (i, 0)),
            pl.BlockSpec((TN, H // 2), lambda i: (i, 0)),
        ],
        out_shape=[
            jax.ShapeDtypeStruct((N, H // 2), jnp.int32),
            jax.ShapeDtypeStruct((N, H // 2), jnp.int32),
        ],
    )(x, W_i, W_c, W_o, bi, bc, bo, w_co, W1a, W1b)


def _pack_tc(t):
    W = t.shape[1] // 2
    lo = lax.bitcast_convert_type(t[:, :W], jnp.int32)
    hi = lax.bitcast_convert_type(t[:, W:], jnp.int32)
    lo16 = lax.shift_right_logical(lo + 0x8000, 16)
    hi16 = (hi + 0x8000) & _HIMASK
    return lo16 | hi16


def _packed_add(a, b):
    f32 = lambda v: lax.bitcast_convert_type(v, jnp.float32)
    i32 = lambda v: lax.bitcast_convert_type(v, jnp.int32)
    lo = i32(f32(a << 16) + f32(b << 16))
    hi = i32(f32(a & _HIMASK) + f32(b & _HIMASK))
    lo16 = lax.shift_right_logical(lo + 0x8000, 16)
    hi16 = (hi + 0x8000) & _HIMASK
    return lo16 | hi16


def _gather_add(A_pk, B_pk, src, dst):
    E = src.shape[0]
    HW = A_pk.shape[1]
    per_w = E // _NW
    n_chunks = per_w // _CH
    n_blocks = n_chunks // _NBUF
    mesh = plsc.VectorSubcoreMesh(core_axis_name="c", subcore_axis_name="s")

    scratch = [
        pltpu.VMEM((per_w,), jnp.int32),
        pltpu.VMEM((per_w,), jnp.int32),
    ]
    scratch += [pltpu.VMEM((_CH, HW), jnp.int32) for _ in range(2 * _NBUF)]
    scratch += [pltpu.VMEM((_CH // 2, 2 * HW), jnp.int32) for _ in range(_NBUF)]
    scratch += [pltpu.SemaphoreType.DMA for _ in range(2 * _NBUF)]

    @functools.partial(
        pl.kernel,
        mesh=mesh,
        out_type=jax.ShapeDtypeStruct((E // 2, 2 * HW), jnp.int32),
        scratch_types=scratch,
        compiler_params=pltpu.CompilerParams(use_tc_tiling_on_sc=False),
    )
    def k(a_hbm, b_hbm, src_hbm, dst_hbm, g_hbm, idx_s, idx_d, *scr):
        bufA = scr[0:_NBUF]
        bufB = scr[_NBUF:2 * _NBUF]
        bufO = scr[2 * _NBUF:3 * _NBUF]
        semg = scr[3 * _NBUF:4 * _NBUF]
        sems = scr[4 * _NBUF:5 * _NBUF]

        wid = lax.axis_index("s") * _NC + lax.axis_index("c")
        w_base2 = wid * (per_w // 2)

        pltpu.sync_copy(src_hbm.at[pl.ds(wid * per_w, per_w)], idx_s)
        pltpu.sync_copy(dst_hbm.at[pl.ds(wid * per_w, per_w)], idx_d)

        def issue_gathers(t, b):
            pltpu.async_copy(
                a_hbm.at[idx_s.at[pl.ds(t * _CH, _CH)]], bufA[b], semg[b])
            pltpu.async_copy(
                b_hbm.at[idx_d.at[pl.ds(t * _CH, _CH)]], bufB[b], semg[b])

        for b in range(_NBUF):
            issue_gathers(b, b)

        def block(g, carry):
            for b in range(_NBUF):
                t = g * _NBUF + b
                pltpu.make_async_copy(
                    a_hbm.at[idx_s.at[pl.ds(0, _CH)]], bufA[b], semg[b]).wait()
                pltpu.make_async_copy(
                    b_hbm.at[idx_d.at[pl.ds(0, _CH)]], bufB[b], semg[b]).wait()

                @pl.when(g >= 1)
                def _():
                    pltpu.make_async_copy(
                        bufO[b], g_hbm.at[pl.ds(0, _CH // 2)], sems[b]).wait()

                def row(p, c2):
                    for u in range(2):
                        r = 2 * p + u
                        for j in range(HW // _LANES):
                            sl = pl.ds(j * _LANES, _LANES)
                            osl = pl.ds(u * HW + j * _LANES, _LANES)
                            bufO[b][p, osl] = _packed_add(
                                bufA[b][r, sl], bufB[b][r, sl])
                    return c2

                lax.fori_loop(0, _CH // 2, row, 0)

                pltpu.async_copy(
                    bufO[b],
                    g_hbm.at[pl.ds(w_base2 + t * (_CH // 2), _CH // 2)],
                    sems[b])

                @pl.when(g < n_blocks - 1)
                def _():
                    issue_gathers(t + _NBUF, b)
            return carry

        lax.fori_loop(0, n_blocks, block, 0)

        for b in range(_NBUF):
            pltpu.make_async_copy(
                bufO[b], g_hbm.at[pl.ds(0, _CH // 2)], sems[b]).wait()

    return k(A_pk, B_pk, src, dst)


def _interleave_out(outT):
    E2 = outT.shape[1]
    per_w = E2 // _NW
    CH3 = 1000
    n3 = per_w // CH3
    mesh = plsc.VectorSubcoreMesh(core_axis_name="c", subcore_axis_name="s")

    @functools.partial(
        pl.kernel,
        mesh=mesh,
        out_type=[
            jax.ShapeDtypeStruct((2 * E2,), jnp.float32),
            jax.ShapeDtypeStruct((2 * E2,), jnp.float32),
        ],
        scratch_types=[
            pltpu.VMEM((4, CH3), jnp.float32),
            pltpu.VMEM((2 * CH3,), jnp.float32),
            pltpu.VMEM((2 * CH3,), jnp.float32),
        ],
        compiler_params=pltpu.CompilerParams(
            use_tc_tiling_on_sc=False, needs_layout_passes=False),
    )
    def k(in_hbm, o0_hbm, o1_hbm, vin, v0, v1):
        wid = lax.axis_index("s") * _NC + lax.axis_index("c")
        pb0 = wid * per_w
        lanes = lax.iota(jnp.int32, _LANES)
        row0 = (lanes & 1) * 2
        row1 = row0 + 1
        col_base = lanes >> 1

        def chunk(t, carry):
            pb = pb0 + t * CH3
            pltpu.sync_copy(in_hbm.at[:, pl.ds(pb, CH3)], vin)

            def grp(g, c2):
                cols = col_base + g * 8
                v0[pl.ds(g * _LANES, _LANES)] = plsc.load_gather(
                    vin, [row0, cols])
                v1[pl.ds(g * _LANES, _LANES)] = plsc.load_gather(
                    vin, [row1, cols])
                return c2

            lax.fori_loop(0, (2 * CH3) // _LANES, grp, 0)
            pltpu.sync_copy(v0, o0_hbm.at[pl.ds(2 * pb, 2 * CH3)])
            pltpu.sync_copy(v1, o1_hbm.at[pl.ds(2 * pb, 2 * CH3)])
            return carry

        lax.fori_loop(0, n3, chunk, 0)

    return k(outT)


def _edge_stage(G2, attr2, W1elo, W1ehi, b1lo, b1hi, W2lo, W2hi, b2T):
    E2, W = G2.shape
    DA = attr2.shape[1]
    TE2 = 16000
    grid = (E2 // TE2,)
    attr3 = attr2.reshape(E2 // TE2, TE2, DA)

    def body(g_ref, attr_hbm, w1elo_ref, w1ehi_ref, b1lo_ref, b1hi_ref,
             w2lo_ref, w2hi_ref, b2t_ref, out_ref, abuf, asem):
        i = pl.program_id(0)
        n = pl.num_programs(0)

        @pl.when(i == 0)
        def _():
            pltpu.make_async_copy(attr_hbm.at[0], abuf.at[0], asem.at[0]).start()

        @pl.when(i + 1 < n)
        def _():
            pltpu.make_async_copy(
                attr_hbm.at[i + 1], abuf.at[(i + 1) % 2],
                asem.at[(i + 1) % 2]).start()

        pltpu.make_async_copy(
            attr_hbm.at[i], abuf.at[i % 2], asem.at[i % 2]).wait()

        bits = g_ref[...]
        glo = lax.bitcast_convert_type(bits << 16, jnp.float32)
        ghi = lax.bitcast_convert_type(bits & _HIMASK, jnp.float32)
        at = abuf[i % 2].astype(jnp.bfloat16)
        elo = jnp.dot(at, w1elo_ref[...], preferred_element_type=jnp.float32)
        ehi = jnp.dot(at, w1ehi_ref[...], preferred_element_type=jnp.float32)
        hid_lo = jnp.maximum(glo + elo + b1lo_ref[...], 0.0).astype(jnp.bfloat16)
        hid_hi = jnp.maximum(ghi + ehi + b1hi_ref[...], 0.0).astype(jnp.bfloat16)
        olo = lax.dot_general(w2lo_ref[...], hid_lo, (((0,), (1,)), ((), ())),
                              preferred_element_type=jnp.float32)
        ohi = lax.dot_general(w2hi_ref[...], hid_hi, (((0,), (1,)), ((), ())),
                              preferred_element_type=jnp.float32)
        out_ref[...] = olo + ohi + b2t_ref[...]

    full = lambda s: pl.BlockSpec(s, lambda i: (0, 0))
    return pl.pallas_call(
        body,
        grid=grid,
        in_specs=[
            pl.BlockSpec((TE2, W), lambda i: (i, 0)),
            pl.BlockSpec(memory_space=pltpu.MemorySpace.HBM),
            full(W1elo.shape), full(W1ehi.shape),
            full(b1lo.shape), full(b1hi.shape),
            full(W2lo.shape), full(W2hi.shape), full(b2T.shape),
        ],
        out_specs=pl.BlockSpec((4, TE2), lambda i: (0, i)),
        out_shape=jax.ShapeDtypeStruct((4, E2), jnp.float32),
        scratch_shapes=[
            pltpu.VMEM((2, TE2, DA), jnp.float32),
            pltpu.SemaphoreType.DMA((2,)),
        ],
    )(G2, attr3, W1elo, W1ehi, b1lo, b1hi, W2lo, W2hi, b2T)


def kernel(x, edge_index, edge_attr, W_i, W_f, W_c, W_o, b_i, b_f, b_c, b_o,
           w_ci, w_cf, w_co, T_i, T_f, T_c, T_o, cb_i, cb_f, cb_c, cb_o,
           W1, b1, W2, b2):
    H = W_i.shape[1]
    Hh = H // 2
    E = edge_index.shape[1]
    bi = b_i + cb_i[None, :]
    bc = b_c + cb_c[None, :]
    bo = b_o + cb_o[None, :]
    W1a = W1[:H]
    W1b = W1[H:2 * H]
    W1e = W1[2 * H:]
    DE = W1e.shape[0]

    Z = jnp.zeros((DE, Hh), dtype=W1e.dtype)
    W1elo = jnp.concatenate([
        jnp.concatenate([W1e[:, :Hh], Z], axis=1),
        jnp.concatenate([Z, W1e[:, :Hh]], axis=1)], axis=0).astype(jnp.bfloat16)
    W1ehi = jnp.concatenate([
        jnp.concatenate([W1e[:, Hh:], Z], axis=1),
        jnp.concatenate([Z, W1e[:, Hh:]], axis=1)], axis=0).astype(jnp.bfloat16)
    b1lo = jnp.concatenate([b1[:Hh], b1[:Hh]])[None, :]
    b1hi = jnp.concatenate([b1[Hh:], b1[Hh:]])[None, :]
    C = W2.shape[1]
    Z2 = jnp.zeros((Hh, C), dtype=W2.dtype)
    W2lo = jnp.concatenate([
        jnp.concatenate([W2[:Hh], Z2], axis=1),
        jnp.concatenate([Z2, W2[:Hh]], axis=1)], axis=0).astype(jnp.bfloat16)
    W2hi = jnp.concatenate([
        jnp.concatenate([W2[Hh:], Z2], axis=1),
        jnp.concatenate([Z2, W2[Hh:]], axis=1)], axis=0).astype(jnp.bfloat16)
    b2T = jnp.concatenate([b2, b2])[:, None]

    A_pk, B_pk = _node_stage(x, W_i, W_c, W_o, bi, bc, bo, w_co, W1a, W1b)
    G2 = _gather_add(A_pk, B_pk, edge_index[0], edge_index[1])
    attr2 = edge_attr.reshape(E // 2, 2 * DE)
    outT = _edge_stage(G2, attr2, W1elo, W1ehi, b1lo, b1hi, W2lo, W2hi, b2T)
    o0, o1 = _interleave_out(outT)
    return jnp.concatenate([o0[:, None], o1[:, None]], axis=1)
